# baseline (speedup 1.0000x reference)
import functools

import jax
import jax.numpy as jnp
from jax import lax
from jax.experimental import pallas as pl
from jax.experimental.pallas import tpu as pltpu
from jax.experimental.pallas import tpu_sc as plsc

N = 100000
E = 1600000

NC = 2
NS = 16
NW = NC * NS
GROUPS = 4
CHUNK = GROUPS * 128
CPT = 100
EPAD = NW * CPT * CHUNK
TRASH = N
STRIPE = 6256
R = NS * STRIPE

BR = 2000
NBLK = N // BR
EPS = 1e-5

_mesh = functools.partial(plsc.VectorSubcoreMesh,
                          core_axis_name="c", subcore_axis_name="s")


def _make_agg(cw):
    @functools.partial(
        pl.kernel,
        out_type=jax.ShapeDtypeStruct((NC, R, cw), jnp.float32),
        mesh=_mesh(),
        compiler_params=pltpu.CompilerParams(use_tc_tiling_on_sc=False),
        scratch_types=[
            pltpu.VMEM((GROUPS, 128), jnp.int32),
            pltpu.VMEM((GROUPS, 128), jnp.int32),
            pltpu.VMEM((GROUPS, 128), jnp.int32),
            pltpu.VMEM((GROUPS, 128), jnp.int32),
            pltpu.VMEM((CHUNK, cw), jnp.float32),
            pltpu.VMEM((CHUNK, cw), jnp.float32),
            pltpu.SemaphoreType.DMA,
            pltpu.SemaphoreType.DMA,
            pltpu.SemaphoreType.DMA,
            pltpu.SemaphoreType.DMA,
            pltpu.VMEM_SHARED((R, cw), jnp.float32),
        ],
    )
    def agg_kernel(hs, src2d, dst2d, zeros, out,
                   src_v0, src_v1, dst_v0, dst_v1, rows_v0, rows_v1,
                   gsem0, gsem1, ssem0, ssem1, acc):
        c = lax.axis_index("c")
        s = lax.axis_index("s")
        t = c * NS + s
        src_vs = (src_v0, src_v1)
        dst_vs = (dst_v0, dst_v1)
        rows_vs = (rows_v0, rows_v1)
        gsems = (gsem0, gsem1)
        ssems = (ssem0, ssem1)
        pltpu.sync_copy(zeros, acc.at[pl.ds(s * STRIPE, STRIPE)])
        plsc.subcore_barrier()
        rowbase = t * CPT * GROUPS

        def load_idx(b, k):
            pltpu.sync_copy(src2d.at[pl.ds(rowbase + k * GROUPS, GROUPS)],
                            src_vs[b])
            pltpu.sync_copy(dst2d.at[pl.ds(rowbase + k * GROUPS, GROUPS)],
                            dst_vs[b])

        def fire_gathers(b):
            for j in range(GROUPS):
                pltpu.async_copy(hs.at[src_vs[b].at[j]],
                                 rows_vs[b].at[pl.ds(j * 128, 128)], gsems[b])

        def drain_gathers(b):
            for j in range(GROUPS):
                pltpu.make_async_copy(hs.at[src_vs[b].at[j]],
                                      rows_vs[b].at[pl.ds(j * 128, 128)],
                                      gsems[b]).wait()

        def fire_scatters(b):
            for j in range(GROUPS):
                pltpu.async_copy(rows_vs[b].at[pl.ds(j * 128, 128)],
                                 acc.at[dst_vs[b].at[j]], ssems[b], add=True)

        def drain_scatters(b):
            for j in range(GROUPS):
                pltpu.make_async_copy(rows_vs[b].at[pl.ds(j * 128, 128)],
                                      acc.at[dst_vs[b].at[j]], ssems[b]).wait()

        load_idx(0, 0)
        fire_gathers(0)

        def step(kk, carry):
            for b in (0, 1):
                k = 2 * kk + b
                drain_gathers(b)
                fire_scatters(b)
                if b == 0:
                    @pl.when(kk > 0)
                    def _():
                        drain_scatters(1)
                else:
                    drain_scatters(0)

                @pl.when(k + 1 < CPT)
                def _():
                    load_idx(1 - b, k + 1)
                    fire_gathers(1 - b)
            return carry

        lax.fori_loop(0, CPT // 2, step, 0)
        drain_scatters(1)
        plsc.subcore_barrier()
        pltpu.sync_copy(acc.at[pl.ds(s * STRIPE, STRIPE)],
                        out.at[c, pl.ds(s * STRIPE, STRIPE)])

    return agg_kernel


def _row_spec(d):
    return pl.BlockSpec((BR, d), lambda i: (i, 0))


def _full_spec(shape):
    nd = len(shape)
    return pl.BlockSpec(shape, lambda i, _nd=nd: (0,) * nd)


def _dinv_spec():
    return pl.BlockSpec((NBLK, BR), lambda i: (0, 0))


def _dv_col(dinv_ref):
    i = pl.program_id(0)
    return jnp.transpose(dinv_ref[pl.ds(i, 1), :])


def _k1_body(dp_ref, x_ref, dinv_ref, xs_ref):
    i = pl.program_id(0)
    dv = lax.rsqrt(dp_ref[0, pl.ds(i, 1), :]
                   + dp_ref[1, pl.ds(i, 1), :] + 1.0)
    dinv_ref[pl.ds(i, 1), :] = dv
    xs_ref[...] = jnp.concatenate(
        [x_ref[...] * jnp.transpose(dv), jnp.zeros((BR, 4), jnp.float32)],
        axis=1)


def _k1(degp3, x):
    return pl.pallas_call(
        _k1_body,
        grid=(NBLK,),
        in_specs=[pl.BlockSpec((NC, NBLK, BR), lambda i: (0, 0, 0)),
                  _row_spec(20)],
        out_specs=[pl.BlockSpec((NBLK, BR), lambda i: (0, 0)), _row_spec(24)],
        out_shape=[jax.ShapeDtypeStruct((NBLK, BR), jnp.float32),
                   jax.ShapeDtypeStruct((N, 24), jnp.float32)],
    )(degp3, x)


def _k2_body(p_ref, xs_ref, dinv_ref, c_ref, s_ref, g_ref, sacc, gacc):
    i = pl.program_id(0)
    p = p_ref[...]
    dv = _dv_col(dinv_ref)
    c = (p[0] + p[1] + xs_ref[...]) * dv
    c_ref[...] = c

    @pl.when(i == 0)
    def _():
        sacc[...] = jnp.zeros_like(sacc)
        gacc[...] = jnp.zeros_like(gacc)

    sacc[...] += jnp.sum(c, axis=0, keepdims=True)
    gacc[...] += lax.dot_general(c, c, (((0,), (0,)), ((), ())),
                                 precision=lax.Precision.HIGHEST,
                                 preferred_element_type=jnp.float32)

    @pl.when(i == NBLK - 1)
    def _():
        s_ref[...] = sacc[...]
        g_ref[...] = gacc[...]


def _k2(P1, xs, dinv):
    return pl.pallas_call(
        _k2_body,
        grid=(NBLK,),
        in_specs=[pl.BlockSpec((NC, BR, 24), lambda i: (0, i, 0)),
                  _row_spec(24), _dinv_spec()],
        out_specs=[_row_spec(24), _full_spec((1, 24)), _full_spec((24, 24))],
        out_shape=[jax.ShapeDtypeStruct((N, 24), jnp.float32),
                   jax.ShapeDtypeStruct((1, 24), jnp.float32),
                   jax.ShapeDtypeStruct((24, 24), jnp.float32)],
        scratch_shapes=[pltpu.VMEM((1, 24), jnp.float32),
                        pltpu.VMEM((24, 24), jnp.float32)],
    )(P1, xs, dinv)


def _k3_body(c_ref, dinv_ref, w1_ref, b1_ref, w2_ref, o_ref):
    h = jnp.maximum(
        jnp.dot(c_ref[...], w1_ref[...], precision=lax.Precision.HIGHEST, preferred_element_type=jnp.float32)
        + b1_ref[...], 0.0)
    o_ref[...] = (jnp.dot(h, w2_ref[...], precision=lax.Precision.HIGHEST, preferred_element_type=jnp.float32)
                  * _dv_col(dinv_ref))


def _k3(c1, dinv, w1e, b1e, w2):
    return pl.pallas_call(
        _k3_body,
        grid=(NBLK,),
        in_specs=[_row_spec(24), _dinv_spec(), _full_spec((24, 64)),
                  _full_spec((1, 64)), _full_spec((64, 48))],
        out_specs=_row_spec(48),
        out_shape=jax.ShapeDtypeStruct((N, 48), jnp.float32),
    )(c1, dinv, w1e, b1e, w2)


def _k4_body(p_ref, ts_ref, dinv_ref, b_ref, z_ref, s_ref, q_ref, sacc, qacc):
    i = pl.program_id(0)
    p = p_ref[...]
    dv = _dv_col(dinv_ref)
    z = (p[0] + p[1] + ts_ref[...]) * dv + b_ref[...]
    z_ref[...] = z

    @pl.when(i == 0)
    def _():
        sacc[...] = jnp.zeros_like(sacc)
        qacc[...] = jnp.zeros_like(qacc)

    sacc[...] += jnp.sum(z, axis=0, keepdims=True)
    qacc[...] += jnp.sum(z * z, axis=0, keepdims=True)

    @pl.when(i == NBLK - 1)
    def _():
        s_ref[...] = sacc[...]
        q_ref[...] = qacc[...]


def _k4(P, ts, dinv, b, d):
    return pl.pallas_call(
        _k4_body,
        grid=(NBLK,),
        in_specs=[pl.BlockSpec((NC, BR, d), lambda i: (0, i, 0)),
                  _row_spec(d), _dinv_spec(), _full_spec((1, d))],
        out_specs=[_row_spec(d), _full_spec((1, d)), _full_spec((1, d))],
        out_shape=[jax.ShapeDtypeStruct((N, d), jnp.float32),
                   jax.ShapeDtypeStruct((1, d), jnp.float32),
                   jax.ShapeDtypeStruct((1, d), jnp.float32)],
        scratch_shapes=[pltpu.VMEM((1, d), jnp.float32),
                        pltpu.VMEM((1, d), jnp.float32)],
    )(P, ts, dinv, b)


def _k5_body(z_ref, sc_ref, sh_ref, w_ref, dinv_ref, o_ref):
    h = jnp.maximum(z_ref[...] * sc_ref[...] + sh_ref[...], 0.0)
    o_ref[...] = (jnp.dot(h, w_ref[...], precision=lax.Precision.HIGHEST, preferred_element_type=jnp.float32)
                  * _dv_col(dinv_ref))


def _k5(z, sc, sh, w, dinv, din, dout):
    return pl.pallas_call(
        _k5_body,
        grid=(NBLK,),
        in_specs=[_row_spec(din), _full_spec((1, din)), _full_spec((1, din)),
                  _full_spec((din, dout)), _dinv_spec()],
        out_specs=_row_spec(dout),
        out_shape=jax.ShapeDtypeStruct((N, dout), jnp.float32),
    )(z, sc, sh, w, dinv)


def _k7_body(z_ref, sc_ref, sh_ref, o_ref):
    o_ref[...] = z_ref[...] * sc_ref[...] + sh_ref[...]


def _k7(z, sc, sh, d):
    return pl.pallas_call(
        _k7_body,
        grid=(NBLK,),
        in_specs=[_row_spec(d), _full_spec((1, d)), _full_spec((1, d))],
        out_specs=_row_spec(d),
        out_shape=jax.ShapeDtypeStruct((N, d), jnp.float32),
    )(z, sc, sh)


def _aggregate(hs, src2d, dst2d, d):
    parts = []
    for c0 in range(0, d, 16):
        cw = min(16, d - c0)
        sl = hs if cw == d else hs[:, c0:c0 + cw]
        zeros = jnp.zeros((STRIPE, cw), jnp.float32)
        p = _make_agg(cw)(sl, src2d, dst2d, zeros)
        parts.append(p[:, :N, :])
    if len(parts) == 1:
        return parts[0]
    return jnp.concatenate(parts, axis=2)


def kernel(x, edge_index, W1, b1, g1, be1, W2, b2, g2, be2, W3, b3, g3, be3):
    src = edge_index[0]
    dst = edge_index[1]
    pad = EPAD - E
    src2d = jnp.concatenate(
        [src, jnp.zeros((pad,), src.dtype)]).reshape(EPAD // 128, 128)
    dst2d = jnp.concatenate(
        [dst, jnp.full((pad,), TRASH, dst.dtype)]).reshape(EPAD // 128, 128)

    degP = _make_agg(16)(jnp.ones((N, 16), jnp.float32), src2d, dst2d,
                         jnp.zeros((STRIPE, 16), jnp.float32))
    degp3 = degP[:, :N, 0].reshape(NC, NBLK, BR)
    dinv, xs = _k1(degp3, x)

    P1 = _aggregate(xs, src2d, dst2d, 24)
    c1, S1, G1 = _k2(P1, xs, dinv)
    W1p = jnp.concatenate([W1, jnp.zeros((4, 64), jnp.float32)], axis=0)
    mu = S1 / N
    M = G1 / N - jnp.transpose(mu) @ mu
    m1 = mu @ W1p + b1[None, :]
    v1 = jnp.sum(W1p * (M @ W1p), axis=0)[None, :]
    s1 = g1[None, :] / jnp.sqrt(v1 + EPS)
    w1e = W1p * s1
    b1e = be1[None, :] + (b1[None, :] - m1) * s1

    ts2 = _k3(c1, dinv, w1e, b1e, W2)
    P2 = _aggregate(ts2, src2d, dst2d, 48)
    z2, S2, Q2 = _k4(P2, ts2, dinv, b2[None, :], 48)
    m2 = S2 / N
    v2 = Q2 / N - m2 * m2
    s2 = g2[None, :] / jnp.sqrt(v2 + EPS)
    sh2 = be2[None, :] - m2 * s2

    ts3 = _k5(z2, s2, sh2, W3, dinv, 48, 32)
    P3 = _aggregate(ts3, src2d, dst2d, 32)
    z3, S3, Q3 = _k4(P3, ts3, dinv, b3[None, :], 32)
    m3 = S3 / N
    v3 = Q3 / N - m3 * m3
    s3 = g3[None, :] / jnp.sqrt(v3 + EPS)
    sh3 = be3[None, :] - m3 * s3
    return _k7(z3, s3, sh3, 32)


# speedup vs baseline: 1.0899x; 1.0899x over previous
import functools

import jax
import jax.numpy as jnp
from jax import lax
from jax.experimental import pallas as pl
from jax.experimental.pallas import tpu as pltpu
from jax.experimental.pallas import tpu_sc as plsc

N = 100000
E = 1600000

NC = 2
NS = 16
NW = NC * NS
GROUPS = 8
CHUNK = GROUPS * 128
CPT = 50
EPAD = NW * CPT * CHUNK
TRASH = N
STRIPE = 6256
R = NS * STRIPE

BR = 2000
NBLK = N // BR
EPS = 1e-5

_mesh = functools.partial(plsc.VectorSubcoreMesh,
                          core_axis_name="c", subcore_axis_name="s")


def _make_agg(cw):
    @functools.partial(
        pl.kernel,
        out_type=jax.ShapeDtypeStruct((NC, R, cw), jnp.float32),
        mesh=_mesh(),
        compiler_params=pltpu.CompilerParams(use_tc_tiling_on_sc=False),
        scratch_types=[
            pltpu.VMEM((GROUPS, 128), jnp.int32),
            pltpu.VMEM((GROUPS, 128), jnp.int32),
            pltpu.VMEM((CHUNK, cw), jnp.float32),
            pltpu.SemaphoreType.DMA,
            pltpu.SemaphoreType.DMA,
            pltpu.VMEM_SHARED((R, cw), jnp.float32),
        ],
    )
    def agg_kernel(hs, src2d, dst2d, zeros, out,
                   src_v, dst_v, rows_v, gsem, ssem, acc):
        c = lax.axis_index("c")
        s = lax.axis_index("s")
        t = c * NS + s
        pltpu.sync_copy(zeros, acc.at[pl.ds(s * STRIPE, STRIPE)])
        plsc.subcore_barrier()
        rowbase = t * CPT * GROUPS

        def chunk(k, carry):
            rb = rowbase + k * GROUPS
            pltpu.sync_copy(src2d.at[pl.ds(rb, GROUPS)], src_v)
            pltpu.sync_copy(dst2d.at[pl.ds(rb, GROUPS)], dst_v)
            gds = [pltpu.async_copy(hs.at[src_v.at[j]],
                                    rows_v.at[pl.ds(j * 128, 128)], gsem)
                   for j in range(GROUPS)]
            for d in gds:
                d.wait()
            sds = [pltpu.async_copy(rows_v.at[pl.ds(j * 128, 128)],
                                    acc.at[dst_v.at[j]], ssem, add=True)
                   for j in range(GROUPS)]
            for d in sds:
                d.wait()
            return carry

        lax.fori_loop(0, CPT, chunk, 0)
        plsc.subcore_barrier()
        pltpu.sync_copy(acc.at[pl.ds(s * STRIPE, STRIPE)],
                        out.at[c, pl.ds(s * STRIPE, STRIPE)])

    return agg_kernel


def _row_spec(d):
    return pl.BlockSpec((BR, d), lambda i: (i, 0))


def _full_spec(shape):
    nd = len(shape)
    return pl.BlockSpec(shape, lambda i, _nd=nd: (0,) * nd)


def _dinv_spec():
    return pl.BlockSpec((NBLK, BR), lambda i: (0, 0))


def _dv_col(dinv_ref):
    i = pl.program_id(0)
    return jnp.transpose(dinv_ref[pl.ds(i, 1), :])


def _k1_body(dp_ref, x_ref, dinv_ref, xs_ref):
    i = pl.program_id(0)
    dv = lax.rsqrt(dp_ref[0, pl.ds(i, 1), :]
                   + dp_ref[1, pl.ds(i, 1), :] + 1.0)
    dinv_ref[pl.ds(i, 1), :] = dv
    xs_ref[...] = jnp.concatenate(
        [x_ref[...] * jnp.transpose(dv), jnp.zeros((BR, 4), jnp.float32)],
        axis=1)


def _k1(degp3, x):
    return pl.pallas_call(
        _k1_body,
        grid=(NBLK,),
        in_specs=[pl.BlockSpec((NC, NBLK, BR), lambda i: (0, 0, 0)),
                  _row_spec(20)],
        out_specs=[pl.BlockSpec((NBLK, BR), lambda i: (0, 0)), _row_spec(24)],
        out_shape=[jax.ShapeDtypeStruct((NBLK, BR), jnp.float32),
                   jax.ShapeDtypeStruct((N, 24), jnp.float32)],
    )(degp3, x)


def _k2_body(p_ref, xs_ref, dinv_ref, c_ref, s_ref, g_ref, sacc, gacc):
    i = pl.program_id(0)
    p = p_ref[...]
    dv = _dv_col(dinv_ref)
    c = (p[0] + p[1] + xs_ref[...]) * dv
    c_ref[...] = c

    @pl.when(i == 0)
    def _():
        sacc[...] = jnp.zeros_like(sacc)
        gacc[...] = jnp.zeros_like(gacc)

    sacc[...] += jnp.sum(c, axis=0, keepdims=True)
    gacc[...] += lax.dot_general(c, c, (((0,), (0,)), ((), ())),
                                 precision=lax.Precision.HIGHEST,
                                 preferred_element_type=jnp.float32)

    @pl.when(i == NBLK - 1)
    def _():
        s_ref[...] = sacc[...]
        g_ref[...] = gacc[...]


def _k2(P1, xs, dinv):
    return pl.pallas_call(
        _k2_body,
        grid=(NBLK,),
        in_specs=[pl.BlockSpec((NC, BR, 24), lambda i: (0, i, 0)),
                  _row_spec(24), _dinv_spec()],
        out_specs=[_row_spec(24), _full_spec((1, 24)), _full_spec((24, 24))],
        out_shape=[jax.ShapeDtypeStruct((N, 24), jnp.float32),
                   jax.ShapeDtypeStruct((1, 24), jnp.float32),
                   jax.ShapeDtypeStruct((24, 24), jnp.float32)],
        scratch_shapes=[pltpu.VMEM((1, 24), jnp.float32),
                        pltpu.VMEM((24, 24), jnp.float32)],
    )(P1, xs, dinv)


def _k3_body(c_ref, dinv_ref, w1_ref, b1_ref, w2_ref, o_ref):
    h = jnp.maximum(
        jnp.dot(c_ref[...], w1_ref[...], precision=lax.Precision.HIGHEST, preferred_element_type=jnp.float32)
        + b1_ref[...], 0.0)
    o_ref[...] = (jnp.dot(h, w2_ref[...], precision=lax.Precision.HIGHEST, preferred_element_type=jnp.float32)
                  * _dv_col(dinv_ref))


def _k3(c1, dinv, w1e, b1e, w2):
    return pl.pallas_call(
        _k3_body,
        grid=(NBLK,),
        in_specs=[_row_spec(24), _dinv_spec(), _full_spec((24, 64)),
                  _full_spec((1, 64)), _full_spec((64, 48))],
        out_specs=_row_spec(48),
        out_shape=jax.ShapeDtypeStruct((N, 48), jnp.float32),
    )(c1, dinv, w1e, b1e, w2)


def _k4_body(p_ref, ts_ref, dinv_ref, b_ref, z_ref, s_ref, q_ref, sacc, qacc):
    i = pl.program_id(0)
    p = p_ref[...]
    dv = _dv_col(dinv_ref)
    z = (p[0] + p[1] + ts_ref[...]) * dv + b_ref[...]
    z_ref[...] = z

    @pl.when(i == 0)
    def _():
        sacc[...] = jnp.zeros_like(sacc)
        qacc[...] = jnp.zeros_like(qacc)

    sacc[...] += jnp.sum(z, axis=0, keepdims=True)
    qacc[...] += jnp.sum(z * z, axis=0, keepdims=True)

    @pl.when(i == NBLK - 1)
    def _():
        s_ref[...] = sacc[...]
        q_ref[...] = qacc[...]


def _k4(P, ts, dinv, b, d):
    return pl.pallas_call(
        _k4_body,
        grid=(NBLK,),
        in_specs=[pl.BlockSpec((NC, BR, d), lambda i: (0, i, 0)),
                  _row_spec(d), _dinv_spec(), _full_spec((1, d))],
        out_specs=[_row_spec(d), _full_spec((1, d)), _full_spec((1, d))],
        out_shape=[jax.ShapeDtypeStruct((N, d), jnp.float32),
                   jax.ShapeDtypeStruct((1, d), jnp.float32),
                   jax.ShapeDtypeStruct((1, d), jnp.float32)],
        scratch_shapes=[pltpu.VMEM((1, d), jnp.float32),
                        pltpu.VMEM((1, d), jnp.float32)],
    )(P, ts, dinv, b)


def _k5_body(z_ref, sc_ref, sh_ref, w_ref, dinv_ref, o_ref):
    h = jnp.maximum(z_ref[...] * sc_ref[...] + sh_ref[...], 0.0)
    o_ref[...] = (jnp.dot(h, w_ref[...], precision=lax.Precision.HIGHEST, preferred_element_type=jnp.float32)
                  * _dv_col(dinv_ref))


def _k5(z, sc, sh, w, dinv, din, dout):
    return pl.pallas_call(
        _k5_body,
        grid=(NBLK,),
        in_specs=[_row_spec(din), _full_spec((1, din)), _full_spec((1, din)),
                  _full_spec((din, dout)), _dinv_spec()],
        out_specs=_row_spec(dout),
        out_shape=jax.ShapeDtypeStruct((N, dout), jnp.float32),
    )(z, sc, sh, w, dinv)


def _k7_body(z_ref, sc_ref, sh_ref, o_ref):
    o_ref[...] = z_ref[...] * sc_ref[...] + sh_ref[...]


def _k7(z, sc, sh, d):
    return pl.pallas_call(
        _k7_body,
        grid=(NBLK,),
        in_specs=[_row_spec(d), _full_spec((1, d)), _full_spec((1, d))],
        out_specs=_row_spec(d),
        out_shape=jax.ShapeDtypeStruct((N, d), jnp.float32),
    )(z, sc, sh)


def _aggregate(hs, src2d, dst2d, d):
    parts = []
    for c0 in range(0, d, 16):
        cw = min(16, d - c0)
        sl = hs if cw == d else hs[:, c0:c0 + cw]
        zeros = jnp.zeros((STRIPE, cw), jnp.float32)
        p = _make_agg(cw)(sl, src2d, dst2d, zeros)
        parts.append(p[:, :N, :])
    if len(parts) == 1:
        return parts[0]
    return jnp.concatenate(parts, axis=2)


def kernel(x, edge_index, W1, b1, g1, be1, W2, b2, g2, be2, W3, b3, g3, be3):
    src = edge_index[0]
    dst = edge_index[1]
    pad = EPAD - E
    src2d = jnp.concatenate(
        [src, jnp.zeros((pad,), src.dtype)]).reshape(EPAD // 128, 128)
    dst2d = jnp.concatenate(
        [dst, jnp.full((pad,), TRASH, dst.dtype)]).reshape(EPAD // 128, 128)

    degP = _make_agg(8)(jnp.ones((N, 8), jnp.float32), src2d, dst2d,
                        jnp.zeros((STRIPE, 8), jnp.float32))
    degp3 = degP[:, :N, 0].reshape(NC, NBLK, BR)
    dinv, xs = _k1(degp3, x)

    P1 = _aggregate(xs, src2d, dst2d, 24)
    c1, S1, G1 = _k2(P1, xs, dinv)
    W1p = jnp.concatenate([W1, jnp.zeros((4, 64), jnp.float32)], axis=0)
    mu = S1 / N
    M = G1 / N - jnp.transpose(mu) @ mu
    m1 = mu @ W1p + b1[None, :]
    v1 = jnp.sum(W1p * (M @ W1p), axis=0)[None, :]
    s1 = g1[None, :] / jnp.sqrt(v1 + EPS)
    w1e = W1p * s1
    b1e = be1[None, :] + (b1[None, :] - m1) * s1

    ts2 = _k3(c1, dinv, w1e, b1e, W2)
    P2 = _aggregate(ts2, src2d, dst2d, 48)
    z2, S2, Q2 = _k4(P2, ts2, dinv, b2[None, :], 48)
    m2 = S2 / N
    v2 = Q2 / N - m2 * m2
    s2 = g2[None, :] / jnp.sqrt(v2 + EPS)
    sh2 = be2[None, :] - m2 * s2

    ts3 = _k5(z2, s2, sh2, W3, dinv, 48, 32)
    P3 = _aggregate(ts3, src2d, dst2d, 32)
    z3, S3, Q3 = _k4(P3, ts3, dinv, b3[None, :], 32)
    m3 = S3 / N
    v3 = Q3 / N - m3 * m3
    s3 = g3[None, :] / jnp.sqrt(v3 + EPS)
    sh3 = be3[None, :] - m3 * s3
    return _k7(z3, s3, sh3, 32)
